# Initial kernel scaffold; baseline (speedup 1.0000x reference)
import jax
import jax.numpy as jnp
from jax.experimental import pallas as pl


def kernel(xyz1, xyz2, xyz2w, points1, points2, w0, b0, w1, b1, wn_w0, wn_b0, wn_w1, wn_b1, wn_w2, wn_b2):
    raise NotImplementedError("write your pallas kernel here")


# speedup vs baseline: 11.9527x; 11.9527x over previous
import functools
import math

import jax
import jax.numpy as jnp
from jax import lax
from jax.experimental import pallas as pl
from jax.experimental.pallas import tpu as pltpu
from jax.experimental.pallas import tpu_sc as plsc

_K = 16
_SQRT_C = math.sqrt(3.0)
_NC = 2
_NS = 16
_NW = _NC * _NS
_HIGH = lax.Precision.HIGHEST


def _leaky(x):
    return jnp.where(x >= 0, x, 0.1 * x)


def _dot(a, b, dims):
    return lax.dot_general(a, b, (dims, ((), ())), precision=_HIGH,
                           preferred_element_type=jnp.float32)


def _knn_body(q_ref, db_ref, idx_ref, *, n2, bn):
    b = pl.program_id(0)
    q = q_ref[0]
    db = db_ref[0]
    qn = jnp.sum(q * q, axis=1, keepdims=True)
    dbn = jnp.sum(db * db, axis=1)[None, :]
    qdb = lax.dot_general(q, db, (((1,), (1,)), ((), ())),
                          preferred_element_type=jnp.float32)
    d = qn + dbn - 2.0 * qdb
    iota = lax.broadcasted_iota(jnp.int32, (bn, n2), 1)
    kio = lax.broadcasted_iota(jnp.int32, (bn, _K), 1)
    off = b * n2
    acc = jnp.zeros((bn, _K), jnp.int32)
    for k in range(_K):
        m = jnp.min(d, axis=1, keepdims=True)
        cand = jnp.where(d == m, iota, n2)
        j = jnp.min(cand, axis=1, keepdims=True)
        acc = jnp.where(kio == k, j + off, acc)
        d = jnp.where(iota == j, jnp.inf, d)
    idx_ref[0] = acc


def _knn(query, db):
    B, N1, _ = query.shape
    N2 = db.shape[1]
    BN = min(256, N1)
    body = functools.partial(_knn_body, n2=N2, bn=BN)
    return pl.pallas_call(
        body,
        grid=(B, N1 // BN),
        in_specs=[
            pl.BlockSpec((1, BN, 3), lambda b, j: (b, j, 0)),
            pl.BlockSpec((1, N2, 3), lambda b, j: (b, 0, 0)),
        ],
        out_specs=pl.BlockSpec((1, BN, _K), lambda b, j: (b, j, 0)),
        out_shape=jax.ShapeDtypeStruct((B, N1, _K), jnp.int32),
    )(query, db)


def _sc_gather(table, idx):
    V, D = table.shape
    Btot = idx.shape[0]
    b_per_w = Btot // _NW
    CH = 128
    nch = b_per_w // CH
    mesh = plsc.VectorSubcoreMesh(core_axis_name="c", subcore_axis_name="s")

    @functools.partial(
        pl.kernel, mesh=mesh,
        out_type=jax.ShapeDtypeStruct((Btot, D), jnp.float32),
        scratch_types=[
            pltpu.VMEM((CH,), jnp.int32),
            pltpu.VMEM((CH, D), jnp.float32),
            pltpu.SemaphoreType.DMA,
        ],
    )
    def gk(table_hbm, idx_hbm, out_hbm, idx_v, rows_v, sem):
        wid = lax.axis_index("s") * _NC + lax.axis_index("c")
        base = wid * b_per_w

        @pl.loop(0, nch)
        def _chunk(i):
            off = base + i * CH
            pltpu.sync_copy(idx_hbm.at[pl.ds(off, CH)], idx_v)
            pltpu.async_copy(table_hbm.at[idx_v], rows_v, sem).wait()
            pltpu.sync_copy(rows_v, out_hbm.at[pl.ds(off, CH)])

    return gk(table, idx)


def _mlp_branch(g3, x1, hp1, w0b, w0c, b0, w1, b1, bn):
    p2g = g3[:, :, :64].reshape(bn * _K, 64)
    dirg = (g3[:, :, 64:67] - x1[:, None, :]).reshape(bn * _K, 3)
    h = _dot(p2g, w0b, ((1,), (1,))) + _dot(dirg, w0c, ((1,), (1,)))
    h = h + jnp.broadcast_to(hp1[:, None, :], (bn, _K, 64)).reshape(bn * _K, 64)
    h = _leaky(h + b0[None, :])
    h = _leaky(_dot(h, w1, ((1,), (1,))) + b1[None, :])
    return h


def _k2_body(p1_ref, x1_ref, ga_ref, gb_ref, w0_ref, b0_ref, w1_ref, b1_ref,
             npo_ref, npwo_ref, qk_ref, *, bn):
    p1 = p1_ref[0]
    x1 = x1_ref[0]
    w0 = w0_ref[...]
    b0 = b0_ref[0]
    w1 = w1_ref[...]
    b1 = b1_ref[0]
    w0a = w0[:, :64]
    w0b = w0[:, 64:128]
    w0c = w0[:, 128:131]
    hp1 = _dot(p1, w0a, ((1,), (1,)))
    np_ = _mlp_branch(ga_ref[...], x1, hp1, w0b, w0c, b0, w1, b1, bn)
    npw = _mlp_branch(gb_ref[...], x1, hp1, w0b, w0c, b0, w1, b1, bn)
    npo_ref[...] = np_
    npwo_ref[...] = npw
    part = lax.dot_general(
        np_.reshape(bn, _K, 64), npw.reshape(bn, _K, 64),
        (((0,), (0,)), ((2,), (2,))), precision=_HIGH,
        preferred_element_type=jnp.float32)

    @pl.when(pl.program_id(1) == 0)
    def _init():
        qk_ref[...] = part

    @pl.when(pl.program_id(1) != 0)
    def _acc():
        qk_ref[...] += part


def _k2(p1t, x1t, ga, gb, w0, b0, w1, b1):
    B, N1, _ = p1t.shape
    BN = min(256, N1)
    nb = N1 // BN
    body = functools.partial(_k2_body, bn=BN)
    rmap = lambda b, j: (b * nb + j, 0)
    rmap3 = lambda b, j: (b * nb + j, 0, 0)
    return pl.pallas_call(
        body,
        grid=(B, nb),
        in_specs=[
            pl.BlockSpec((1, BN, 64), lambda b, j: (b, j, 0)),
            pl.BlockSpec((1, BN, 3), lambda b, j: (b, j, 0)),
            pl.BlockSpec((BN, _K, 128), rmap3),
            pl.BlockSpec((BN, _K, 128), rmap3),
            pl.BlockSpec((64, 131), lambda b, j: (0, 0)),
            pl.BlockSpec((1, 64), lambda b, j: (0, 0)),
            pl.BlockSpec((64, 64), lambda b, j: (0, 0)),
            pl.BlockSpec((1, 64), lambda b, j: (0, 0)),
        ],
        out_specs=[
            pl.BlockSpec((BN * _K, 64), rmap),
            pl.BlockSpec((BN * _K, 64), rmap),
            pl.BlockSpec((64, _K, _K), lambda b, j: (b, 0, 0)),
        ],
        out_shape=[
            jax.ShapeDtypeStruct((B * N1 * _K, 64), jnp.float32),
            jax.ShapeDtypeStruct((B * N1 * _K, 64), jnp.float32),
            jax.ShapeDtypeStruct((B * 64, _K, _K), jnp.float32),
        ],
    )(p1t, x1t, ga, gb, w0, b0.reshape(1, 64), w1, b1.reshape(1, 64))


def _k3_body(qk_ref, npo_ref, npwo_ref, out_ref, *, bn):
    qk = qk_ref[...]
    m = jnp.max(qk, axis=-1, keepdims=True)
    e = jnp.exp(qk - m)
    p = e / jnp.sum(e, axis=-1, keepdims=True)
    p = jnp.maximum(p / _SQRT_C, 1e-10)
    wrow = jnp.transpose(jnp.sum(p, axis=2))
    wcol = jnp.transpose(jnp.sum(p, axis=1))
    np3 = npo_ref[...].reshape(bn, _K, 64)
    npw3 = npwo_ref[...].reshape(bn, _K, 64)
    cost = jnp.sum(np3 * wcol[None, :, :], axis=1)
    costw = jnp.sum(npw3 * wrow[None, :, :], axis=1)
    out_ref[...] = jnp.concatenate([cost, costw], axis=1)


def _k3(qk, npo, npwo, B, N1):
    BN = min(512, N1)
    nb = N1 // BN
    body = functools.partial(_k3_body, bn=BN)
    rmap = lambda b, j: (b * nb + j, 0)
    return pl.pallas_call(
        body,
        grid=(B, nb),
        in_specs=[
            pl.BlockSpec((64, _K, _K), lambda b, j: (b, 0, 0)),
            pl.BlockSpec((BN * _K, 64), rmap),
            pl.BlockSpec((BN * _K, 64), rmap),
        ],
        out_specs=pl.BlockSpec((BN, 128), rmap),
        out_shape=jax.ShapeDtypeStruct((B * N1, 128), jnp.float32),
    )(qk, npo, npwo)


def _w1_body(gst_ref, x1rt_ref, wn0_ref, wb0_ref, wn1_ref, wb1_ref, wn2_ref,
             wb2_ref, y2_ref, st_ref, *, n):
    dirt = gst_ref[...] - x1rt_ref[...]
    wn0 = wn0_ref[...]
    wn1 = wn1_ref[...]
    wn2 = wn2_ref[...]
    z1 = _dot(wn0, dirt, ((1,), (0,))) + wb0_ref[...]
    m1 = jnp.mean(z1, axis=1, keepdims=True)
    v1 = jnp.mean((z1 - m1) * (z1 - m1), axis=1, keepdims=True)
    y1 = jax.nn.relu((z1 - m1) / jnp.sqrt(v1 + 1e-5))
    z2 = _dot(wn1, y1, ((1,), (0,))) + wb1_ref[...]
    m2 = jnp.mean(z2, axis=1, keepdims=True)
    v2 = jnp.mean((z2 - m2) * (z2 - m2), axis=1, keepdims=True)
    y2 = jax.nn.relu((z2 - m2) / jnp.sqrt(v2 + 1e-5))
    y2_ref[...] = y2
    my2 = jnp.mean(y2, axis=1, keepdims=True)
    s2 = _dot(y2, y2, ((1,), (1,))) / float(n)
    m3 = jnp.transpose(_dot(wn2, my2, ((1,), (0,)))) + wb2_ref[...]
    w2s = _dot(wn2, s2, ((1,), (0,)))
    v3 = jnp.sum(w2s * wn2, axis=1)[None, :] - m3 * m3
    srow = lax.broadcasted_iota(jnp.int32, (8, 64), 0)
    st_ref[...] = jnp.where(srow == 0, m3,
                            jnp.where(srow == 1, v3, 0.0))


def _w1(gst, x1rt, wn_w0, wn_b0, wn_w1, wn_b1, wn_w2, wn_b2):
    n = gst.shape[1]
    body = functools.partial(_w1_body, n=n)
    return pl.pallas_call(
        body,
        out_shape=[
            jax.ShapeDtypeStruct((8, n), jnp.float32),
            jax.ShapeDtypeStruct((8, 64), jnp.float32),
        ],
    )(gst, x1rt, wn_w0, wn_b0.reshape(8, 1), wn_w1, wn_b1.reshape(8, 1),
      wn_w2, wn_b2.reshape(1, 64))


def _w2_body(y2_ref, gc_ref, st_ref, wn2_ref, wb2_ref, o1_ref, o2_ref, *, bn):
    y2 = y2_ref[...]
    gc = gc_ref[...]
    st = st_ref[...]
    z3 = _dot(wn2_ref[...], y2, ((1,), (0,)))
    z3 = jnp.transpose(z3) + wb2_ref[...]
    w = jax.nn.relu((z3 - st[0:1, :]) / jnp.sqrt(st[1:2, :] + 1e-5))
    o1_ref[...] = jnp.sum((w * gc[:, :64]).reshape(bn, _K, 64), axis=1)
    o2_ref[...] = jnp.sum((w * gc[:, 64:]).reshape(bn, _K, 64), axis=1)


def _w2(y2, gc, stats, wn_w2, wn_b2):
    rows = y2.shape[1] // _K
    BN = min(512, rows)
    body = functools.partial(_w2_body, bn=BN)
    return pl.pallas_call(
        body,
        grid=(rows // BN,),
        in_specs=[
            pl.BlockSpec((8, BN * _K), lambda i: (0, i)),
            pl.BlockSpec((BN * _K, 128), lambda i: (i, 0)),
            pl.BlockSpec((8, 64), lambda i: (0, 0)),
            pl.BlockSpec((64, 8), lambda i: (0, 0)),
            pl.BlockSpec((1, 64), lambda i: (0, 0)),
        ],
        out_specs=[
            pl.BlockSpec((BN, 64), lambda i: (i, 0)),
            pl.BlockSpec((BN, 64), lambda i: (i, 0)),
        ],
        out_shape=[
            jax.ShapeDtypeStruct((rows, 64), jnp.float32),
            jax.ShapeDtypeStruct((rows, 64), jnp.float32),
        ],
    )(y2, gc, stats, wn_w2, wn_b2.reshape(1, 64))


def kernel(xyz1, xyz2, xyz2w, points1, points2, w0, b0, w1, b1,
           wn_w0, wn_b0, wn_w1, wn_b1, wn_w2, wn_b2):
    B, _, N1 = xyz1.shape
    N2 = xyz2.shape[2]
    D = points1.shape[1]
    M = w0.shape[0]
    x1t = jnp.transpose(xyz1, (0, 2, 1))
    x2t = jnp.transpose(xyz2, (0, 2, 1))
    x2wt = jnp.transpose(xyz2w, (0, 2, 1))
    p1t = jnp.transpose(points1, (0, 2, 1))
    p2t = jnp.transpose(points2, (0, 2, 1))

    idx_a = _knn(x1t, x2t)
    idx_b = _knn(x1t, x2wt)
    idx_s = _knn(x1t, x1t)

    pad = jnp.zeros((B * N2, 61), jnp.float32)
    p2f = p2t.reshape(B * N2, D)
    tab_a = jnp.concatenate([p2f, x2t.reshape(B * N2, 3), pad], axis=1)
    tab_b = jnp.concatenate([p2f, x2wt.reshape(B * N2, 3), pad], axis=1)
    tab_s = jnp.concatenate(
        [x1t.reshape(B * N1, 3), jnp.zeros((B * N1, 125), jnp.float32)], axis=1)

    ga = _sc_gather(tab_a, idx_a.reshape(-1))
    gb = _sc_gather(tab_b, idx_b.reshape(-1))
    gs = _sc_gather(tab_s, idx_s.reshape(-1))
    gst = jnp.transpose(gs[:, :3])

    npo, npwo, qk = _k2(p1t, x1t, ga.reshape(B * N1, _K, 128),
                        gb.reshape(B * N1, _K, 128), w0, b0, w1, b1)
    cost_tab = _k3(qk, npo, npwo, B, N1)

    x1rt = jnp.repeat(jnp.transpose(x1t.reshape(B * N1, 3)), _K, axis=1)
    y2, stats = _w1(gst, x1rt, wn_w0, wn_b0, wn_w1, wn_b1, wn_w2, wn_b2)

    gc = _sc_gather(cost_tab, idx_s.reshape(-1))
    pc, pcw = _w2(y2, gc, stats, wn_w2, wn_b2)
    pc = jnp.transpose(pc.reshape(B, N1, M), (0, 2, 1))
    pcw = jnp.transpose(pcw.reshape(B, N1, M), (0, 2, 1))
    return pc, pcw


# speedup vs baseline: 12.4306x; 1.0400x over previous
import functools
import math

import jax
import jax.numpy as jnp
from jax import lax
from jax.experimental import pallas as pl
from jax.experimental.pallas import tpu as pltpu
from jax.experimental.pallas import tpu_sc as plsc

_K = 16
_SQRT_C = math.sqrt(3.0)
_NC = 2
_NS = 16
_NW = _NC * _NS
_HIGH = lax.Precision.HIGHEST


def _leaky(x):
    return jnp.where(x >= 0, x, 0.1 * x)


def _dot(a, b, dims):
    return lax.dot_general(a, b, (dims, ((), ())), precision=_HIGH,
                           preferred_element_type=jnp.float32)


def _knn_body(q_ref, db_ref, idx_ref, *, n2, bn):
    b = pl.program_id(0)
    q = q_ref[0]
    db = db_ref[0]
    qn = jnp.sum(q * q, axis=1, keepdims=True)
    dbn = jnp.sum(db * db, axis=1)[None, :]
    qdb = lax.dot_general(q, db, (((1,), (1,)), ((), ())),
                          preferred_element_type=jnp.float32)
    d = qn + dbn - 2.0 * qdb
    iota = lax.broadcasted_iota(jnp.int32, (bn, n2), 1)
    kio = lax.broadcasted_iota(jnp.int32, (bn, _K), 1)
    off = b * n2
    acc = jnp.zeros((bn, _K), jnp.int32)
    for k in range(_K):
        m = jnp.min(d, axis=1, keepdims=True)
        cand = jnp.where(d == m, iota, n2)
        j = jnp.min(cand, axis=1, keepdims=True)
        acc = jnp.where(kio == k, j + off, acc)
        d = jnp.where(iota == j, jnp.inf, d)
    idx_ref[0] = acc


def _knn(query, db):
    B, N1, _ = query.shape
    N2 = db.shape[1]
    BN = min(512, N1)
    body = functools.partial(_knn_body, n2=N2, bn=BN)
    return pl.pallas_call(
        body,
        grid=(B, N1 // BN),
        in_specs=[
            pl.BlockSpec((1, BN, 3), lambda b, j: (b, j, 0)),
            pl.BlockSpec((1, N2, 3), lambda b, j: (b, 0, 0)),
        ],
        out_specs=pl.BlockSpec((1, BN, _K), lambda b, j: (b, j, 0)),
        out_shape=jax.ShapeDtypeStruct((B, N1, _K), jnp.int32),
    )(query, db)


def _sc_gather(table, idx):
    V, D = table.shape
    Btot = idx.shape[0]
    b_per_w = Btot // _NW
    CH = 128
    nch = b_per_w // CH
    mesh = plsc.VectorSubcoreMesh(core_axis_name="c", subcore_axis_name="s")

    @functools.partial(
        pl.kernel, mesh=mesh,
        out_type=jax.ShapeDtypeStruct((Btot, D), jnp.float32),
        scratch_types=[
            pltpu.VMEM((CH,), jnp.int32),
            pltpu.VMEM((CH, D), jnp.float32),
            pltpu.SemaphoreType.DMA,
        ],
    )
    def gk(table_hbm, idx_hbm, out_hbm, idx_v, rows_v, sem):
        wid = lax.axis_index("s") * _NC + lax.axis_index("c")
        base = wid * b_per_w

        @pl.loop(0, nch)
        def _chunk(i):
            off = base + i * CH
            pltpu.sync_copy(idx_hbm.at[pl.ds(off, CH)], idx_v)
            pltpu.async_copy(table_hbm.at[idx_v], rows_v, sem).wait()
            pltpu.sync_copy(rows_v, out_hbm.at[pl.ds(off, CH)])

    return gk(table, idx)


def _mlp_branch(g3, x1, hp1, w0b, w0c, b0, w1, b1, bn):
    p2g = g3[:, :, :64].reshape(bn * _K, 64)
    dirg = (g3[:, :, 64:67] - x1[:, None, :]).reshape(bn * _K, 3)
    h = _dot(p2g, w0b, ((1,), (1,))) + _dot(dirg, w0c, ((1,), (1,)))
    h = h + jnp.broadcast_to(hp1[:, None, :], (bn, _K, 64)).reshape(bn * _K, 64)
    h = _leaky(h + b0[None, :])
    h = _leaky(_dot(h, w1, ((1,), (1,))) + b1[None, :])
    return h


def _k2_body(p1_ref, x1_ref, ga_ref, gb_ref, w0_ref, b0_ref, w1_ref, b1_ref,
             npo_ref, npwo_ref, qk_ref, *, bn):
    p1 = p1_ref[0]
    x1 = x1_ref[0]
    w0 = w0_ref[...]
    b0 = b0_ref[0]
    w1 = w1_ref[...]
    b1 = b1_ref[0]
    w0a = w0[:, :64]
    w0b = w0[:, 64:128]
    w0c = w0[:, 128:131]
    hp1 = _dot(p1, w0a, ((1,), (1,)))
    np_ = _mlp_branch(ga_ref[...], x1, hp1, w0b, w0c, b0, w1, b1, bn)
    npw = _mlp_branch(gb_ref[...], x1, hp1, w0b, w0c, b0, w1, b1, bn)
    npo_ref[...] = np_
    npwo_ref[...] = npw
    part = lax.dot_general(
        np_.reshape(bn, _K, 64), npw.reshape(bn, _K, 64),
        (((0,), (0,)), ((2,), (2,))), precision=_HIGH,
        preferred_element_type=jnp.float32)

    @pl.when(pl.program_id(1) == 0)
    def _init():
        qk_ref[...] = part

    @pl.when(pl.program_id(1) != 0)
    def _acc():
        qk_ref[...] += part


def _k2(p1t, x1t, ga, gb, w0, b0, w1, b1):
    B, N1, _ = p1t.shape
    BN = min(256, N1)
    nb = N1 // BN
    body = functools.partial(_k2_body, bn=BN)
    rmap = lambda b, j: (b * nb + j, 0)
    rmap3 = lambda b, j: (b * nb + j, 0, 0)
    return pl.pallas_call(
        body,
        grid=(B, nb),
        in_specs=[
            pl.BlockSpec((1, BN, 64), lambda b, j: (b, j, 0)),
            pl.BlockSpec((1, BN, 3), lambda b, j: (b, j, 0)),
            pl.BlockSpec((BN, _K, 128), rmap3),
            pl.BlockSpec((BN, _K, 128), rmap3),
            pl.BlockSpec((64, 131), lambda b, j: (0, 0)),
            pl.BlockSpec((1, 64), lambda b, j: (0, 0)),
            pl.BlockSpec((64, 64), lambda b, j: (0, 0)),
            pl.BlockSpec((1, 64), lambda b, j: (0, 0)),
        ],
        out_specs=[
            pl.BlockSpec((BN * _K, 64), rmap),
            pl.BlockSpec((BN * _K, 64), rmap),
            pl.BlockSpec((64, _K, _K), lambda b, j: (b, 0, 0)),
        ],
        out_shape=[
            jax.ShapeDtypeStruct((B * N1 * _K, 64), jnp.float32),
            jax.ShapeDtypeStruct((B * N1 * _K, 64), jnp.float32),
            jax.ShapeDtypeStruct((B * 64, _K, _K), jnp.float32),
        ],
    )(p1t, x1t, ga, gb, w0, b0.reshape(1, 64), w1, b1.reshape(1, 64))


def _k3_body(qk_ref, npo_ref, npwo_ref, out_ref, *, bn):
    qk = qk_ref[...]
    m = jnp.max(qk, axis=-1, keepdims=True)
    e = jnp.exp(qk - m)
    p = e / jnp.sum(e, axis=-1, keepdims=True)
    p = jnp.maximum(p / _SQRT_C, 1e-10)
    wrow = jnp.transpose(jnp.sum(p, axis=2))
    wcol = jnp.transpose(jnp.sum(p, axis=1))
    np3 = npo_ref[...].reshape(bn, _K, 64)
    npw3 = npwo_ref[...].reshape(bn, _K, 64)
    cost = jnp.sum(np3 * wcol[None, :, :], axis=1)
    costw = jnp.sum(npw3 * wrow[None, :, :], axis=1)
    out_ref[...] = jnp.concatenate([cost, costw], axis=1)


def _k3(qk, npo, npwo, B, N1):
    BN = min(512, N1)
    nb = N1 // BN
    body = functools.partial(_k3_body, bn=BN)
    rmap = lambda b, j: (b * nb + j, 0)
    return pl.pallas_call(
        body,
        grid=(B, nb),
        in_specs=[
            pl.BlockSpec((64, _K, _K), lambda b, j: (b, 0, 0)),
            pl.BlockSpec((BN * _K, 64), rmap),
            pl.BlockSpec((BN * _K, 64), rmap),
        ],
        out_specs=pl.BlockSpec((BN, 128), rmap),
        out_shape=jax.ShapeDtypeStruct((B * N1, 128), jnp.float32),
    )(qk, npo, npwo)


def _w1_body(gst_ref, x1rt_ref, wn0_ref, wb0_ref, wn1_ref, wb1_ref, wn2_ref,
             wb2_ref, y2_ref, st_ref, *, n):
    dirt = gst_ref[...] - x1rt_ref[...]
    wn0 = wn0_ref[...]
    wn1 = wn1_ref[...]
    wn2 = wn2_ref[...]
    z1 = _dot(wn0, dirt, ((1,), (0,))) + wb0_ref[...]
    m1 = jnp.mean(z1, axis=1, keepdims=True)
    v1 = jnp.mean((z1 - m1) * (z1 - m1), axis=1, keepdims=True)
    y1 = jax.nn.relu((z1 - m1) / jnp.sqrt(v1 + 1e-5))
    z2 = _dot(wn1, y1, ((1,), (0,))) + wb1_ref[...]
    m2 = jnp.mean(z2, axis=1, keepdims=True)
    v2 = jnp.mean((z2 - m2) * (z2 - m2), axis=1, keepdims=True)
    y2 = jax.nn.relu((z2 - m2) / jnp.sqrt(v2 + 1e-5))
    y2_ref[...] = y2
    my2 = jnp.mean(y2, axis=1, keepdims=True)
    s2 = _dot(y2, y2, ((1,), (1,))) / float(n)
    m3 = jnp.transpose(_dot(wn2, my2, ((1,), (0,)))) + wb2_ref[...]
    w2s = _dot(wn2, s2, ((1,), (0,)))
    v3 = jnp.sum(w2s * wn2, axis=1)[None, :] - m3 * m3
    srow = lax.broadcasted_iota(jnp.int32, (8, 64), 0)
    st_ref[...] = jnp.where(srow == 0, m3,
                            jnp.where(srow == 1, v3, 0.0))


def _w1(gst, x1rt, wn_w0, wn_b0, wn_w1, wn_b1, wn_w2, wn_b2):
    n = gst.shape[1]
    body = functools.partial(_w1_body, n=n)
    return pl.pallas_call(
        body,
        out_shape=[
            jax.ShapeDtypeStruct((8, n), jnp.float32),
            jax.ShapeDtypeStruct((8, 64), jnp.float32),
        ],
    )(gst, x1rt, wn_w0, wn_b0.reshape(8, 1), wn_w1, wn_b1.reshape(8, 1),
      wn_w2, wn_b2.reshape(1, 64))


def _w2_body(y2_ref, gc_ref, st_ref, wn2_ref, wb2_ref, o1_ref, o2_ref, *, bn):
    y2 = y2_ref[...]
    gc = gc_ref[...]
    st = st_ref[...]
    z3 = _dot(wn2_ref[...], y2, ((1,), (0,)))
    z3 = jnp.transpose(z3) + wb2_ref[...]
    w = jax.nn.relu((z3 - st[0:1, :]) / jnp.sqrt(st[1:2, :] + 1e-5))
    o1_ref[...] = jnp.sum((w * gc[:, :64]).reshape(bn, _K, 64), axis=1)
    o2_ref[...] = jnp.sum((w * gc[:, 64:]).reshape(bn, _K, 64), axis=1)


def _w2(y2, gc, stats, wn_w2, wn_b2):
    rows = y2.shape[1] // _K
    BN = min(512, rows)
    body = functools.partial(_w2_body, bn=BN)
    return pl.pallas_call(
        body,
        grid=(rows // BN,),
        in_specs=[
            pl.BlockSpec((8, BN * _K), lambda i: (0, i)),
            pl.BlockSpec((BN * _K, 128), lambda i: (i, 0)),
            pl.BlockSpec((8, 64), lambda i: (0, 0)),
            pl.BlockSpec((64, 8), lambda i: (0, 0)),
            pl.BlockSpec((1, 64), lambda i: (0, 0)),
        ],
        out_specs=[
            pl.BlockSpec((BN, 64), lambda i: (i, 0)),
            pl.BlockSpec((BN, 64), lambda i: (i, 0)),
        ],
        out_shape=[
            jax.ShapeDtypeStruct((rows, 64), jnp.float32),
            jax.ShapeDtypeStruct((rows, 64), jnp.float32),
        ],
    )(y2, gc, stats, wn_w2, wn_b2.reshape(1, 64))


def kernel(xyz1, xyz2, xyz2w, points1, points2, w0, b0, w1, b1,
           wn_w0, wn_b0, wn_w1, wn_b1, wn_w2, wn_b2):
    B, _, N1 = xyz1.shape
    N2 = xyz2.shape[2]
    D = points1.shape[1]
    M = w0.shape[0]
    x1t = jnp.transpose(xyz1, (0, 2, 1))
    x2t = jnp.transpose(xyz2, (0, 2, 1))
    x2wt = jnp.transpose(xyz2w, (0, 2, 1))
    p1t = jnp.transpose(points1, (0, 2, 1))
    p2t = jnp.transpose(points2, (0, 2, 1))

    idx_a = _knn(x1t, x2t)
    idx_b = _knn(x1t, x2wt)
    idx_s = _knn(x1t, x1t)

    pad = jnp.zeros((B * N2, 61), jnp.float32)
    p2f = p2t.reshape(B * N2, D)
    tab_a = jnp.concatenate([p2f, x2t.reshape(B * N2, 3), pad], axis=1)
    tab_b = jnp.concatenate([p2f, x2wt.reshape(B * N2, 3), pad], axis=1)
    tab_s = jnp.concatenate(
        [x1t.reshape(B * N1, 3), jnp.zeros((B * N1, 125), jnp.float32)], axis=1)

    ga = _sc_gather(tab_a, idx_a.reshape(-1))
    gb = _sc_gather(tab_b, idx_b.reshape(-1))
    gs = _sc_gather(tab_s, idx_s.reshape(-1))
    gst = jnp.transpose(gs[:, :3])

    npo, npwo, qk = _k2(p1t, x1t, ga.reshape(B * N1, _K, 128),
                        gb.reshape(B * N1, _K, 128), w0, b0, w1, b1)
    cost_tab = _k3(qk, npo, npwo, B, N1)

    x1rt = jnp.repeat(jnp.transpose(x1t.reshape(B * N1, 3)), _K, axis=1)
    y2, stats = _w1(gst, x1rt, wn_w0, wn_b0, wn_w1, wn_b1, wn_w2, wn_b2)

    gc = _sc_gather(cost_tab, idx_s.reshape(-1))
    pc, pcw = _w2(y2, gc, stats, wn_w2, wn_b2)
    pc = jnp.transpose(pc.reshape(B, N1, M), (0, 2, 1))
    pcw = jnp.transpose(pcw.reshape(B, N1, M), (0, 2, 1))
    return pc, pcw
